# Initial kernel scaffold; baseline (speedup 1.0000x reference)
import jax
import jax.numpy as jnp
from jax.experimental import pallas as pl


def kernel(x, dependency_triples, W_self, b_self, W_dep, b_dep):
    raise NotImplementedError("write your pallas kernel here")


# speedup vs baseline: 1.2374x; 1.2374x over previous
import functools

import jax
import jax.numpy as jnp
from jax import lax
from jax.experimental import pallas as pl
from jax.experimental.pallas import tpu as pltpu
from jax.experimental.pallas import tpu_sc as plsc

N = 20000
D = 64
E = 20000
L = 40

NC, NS = 2, 16
NW = NC * NS
CH = 128

EP = 20480
EP2 = 2 * EP
ROWS_G = EP2 // NW
KG = ROWS_G // CH
ROWS_S = EP2 // NS
KS = ROWS_S // CH
TM = 1024
GRID = EP2 // TM
GRID_BASE = EP // TM

@functools.cache
def _sc_kernels():
    mesh = plsc.VectorSubcoreMesh(core_axis_name="c", subcore_axis_name="s")

    @functools.partial(
        pl.kernel,
        mesh=mesh,
        out_type=jax.ShapeDtypeStruct((EP2, 2 * D), jnp.float32),
        scratch_types=[
            pltpu.VMEM((KG, CH), jnp.int32),
            pltpu.VMEM((CH, 2 * D), jnp.float32),
            pltpu.VMEM((CH, 2 * D), jnp.float32),
            pltpu.SemaphoreType.DMA,
            pltpu.SemaphoreType.DMA,
        ],
    )
    def sc_gather(x_hbm, idx_hbm, g_hbm, idx_v, buf0, buf1, sem0, sem1):
        wid = lax.axis_index("s") * NC + lax.axis_index("c")
        pltpu.sync_copy(idx_hbm.at[wid], idx_v)
        bufs, sems, descs = (buf0, buf1), (sem0, sem1), [None, None]
        descs[0] = pltpu.async_copy(x_hbm.at[idx_v.at[0]], buf0, sem0)
        for j in range(KG):
            sl = j % 2
            if j + 1 < KG:
                descs[1 - sl] = pltpu.async_copy(
                    x_hbm.at[idx_v.at[j + 1]], bufs[1 - sl], sems[1 - sl])
            descs[sl].wait()
            pltpu.sync_copy(bufs[sl],
                            g_hbm.at[pl.ds(wid * ROWS_G + j * CH, CH)])

    rows_i = EP // NS
    rows_m = EP2 // NW
    km = rows_m // CH

    rows_o = EP // NW
    n_chunks = EP2 // CH

    @functools.partial(
        pl.kernel,
        mesh=mesh,
        out_type=jax.ShapeDtypeStruct((EP, D), jnp.float32),
        scratch_types=[
            pltpu.VMEM((CH,), jnp.int32),
            pltpu.VMEM((CH, D), jnp.float32),
            pltpu.VMEM_SHARED((NS * (rows_o + 16), D), jnp.float32),
        ],
    )
    def sc_scatter(init_hbm, msg_hbm, sidx_hbm, out_hbm,
                   lidx_v, msg_v, acc_s):
        s = lax.axis_index("s")
        wid = s * NC + lax.axis_index("c")
        lo = wid * rows_o
        a0 = s * (rows_o + 16)
        for j in range(rows_o // CH):
            pltpu.sync_copy(init_hbm.at[pl.ds(lo + j * CH, CH)], msg_v)
            pltpu.sync_copy(msg_v, acc_s.at[pl.ds(a0 + j * CH, CH)])

        def chunk_body(j, carry):
            pltpu.sync_copy(sidx_hbm.at[pl.ds((wid * n_chunks + j) * CH, CH)],
                            lidx_v)
            pltpu.sync_copy(msg_hbm.at[pl.ds(j * CH, CH)], msg_v)
            pltpu.sync_copy(msg_v, acc_s.at[lidx_v], add=True)
            return carry

        lax.fori_loop(0, n_chunks, chunk_body, 0)
        for j in range(rows_o // CH):
            pltpu.sync_copy(acc_s.at[pl.ds(a0 + j * CH, CH)], msg_v)
            pltpu.sync_copy(msg_v, out_hbm.at[pl.ds(lo + j * CH, CH)])

    return sc_gather, sc_scatter


def _local_scatter_indices(scat_idx):
    rows_o = EP // NW
    w = jnp.arange(NW, dtype=jnp.int32)
    lo = w * rows_o
    a0 = (w // NC) * (rows_o + 16)
    local = scat_idx[None, :] - lo[:, None]
    ok = (local >= 0) & (local < rows_o)
    return (a0[:, None] + jnp.where(ok, local, rows_o)).reshape(-1)


def _tc_body(lab_ref, g_ref, x_ref, wt_ref, b2_ref, wself_ref, bself_ref,
             msg_ref, base_ref):
    i = pl.program_id(0)
    g = g_ref[...]
    lab = lab_ref[...]
    oh = (lab == lax.broadcasted_iota(jnp.int32, (1, L), 1)
          ).astype(jnp.float32)
    acc = jnp.dot(oh, b2_ref[0], preferred_element_type=jnp.float32)
    h = jnp.dot(g, wt_ref[0], preferred_element_type=jnp.float32)
    labh = lab // 2
    labp = lab % 2
    jhalf = lax.broadcasted_iota(jnp.int32, (1, 2 * D), 1) // D
    acc128 = jnp.zeros((TM, 2 * D), jnp.float32)
    for p in range(L // 2):
        m = ((labh == p) & (jhalf == labp)).astype(jnp.float32)
        acc128 = acc128 + m * h[:, 2 * D * p:2 * D * (p + 1)]
    acc = acc + acc128[:, :D] + acc128[:, D:]
    msg_ref[...] = acc

    @pl.when(i < GRID_BASE)
    def _():
        base_ref[...] = (
            jnp.dot(x_ref[...], wself_ref[...],
                    preferred_element_type=jnp.float32) + bself_ref[...])


def _tc_compute(lab2, g, x_pad, wt2d, b2, wself_t, bself2d):
    return pl.pallas_call(
        _tc_body,
        grid=(GRID,),
        in_specs=[
            pl.BlockSpec((TM, 1), lambda i: (i, 0)),
            pl.BlockSpec((TM, 2 * D), lambda i: (i, 0)),
            pl.BlockSpec((TM, D), lambda i: (jnp.minimum(i, GRID_BASE - 1), 0)),
            pl.BlockSpec((1, 2 * D, L * D), lambda i: (i // GRID_BASE, 0, 0)),
            pl.BlockSpec((1, L, D), lambda i: (i // GRID_BASE, 0, 0)),
            pl.BlockSpec((D, D), lambda i: (0, 0)),
            pl.BlockSpec((1, D), lambda i: (0, 0)),
        ],
        out_specs=[
            pl.BlockSpec((TM, D), lambda i: (i, 0)),
            pl.BlockSpec((TM, D), lambda i: (jnp.minimum(i, GRID_BASE - 1), 0)),
        ],
        out_shape=[
            jax.ShapeDtypeStruct((EP2, D), jnp.float32),
            jax.ShapeDtypeStruct((EP, D), jnp.float32),
        ],
    )(lab2, g, x_pad, wt2d, b2, wself_t, bself2d)


def _tc_scatter_body(idx_ref, msg_ref, base_ref, o_ref):
    i = pl.program_id(0)
    j = pl.program_id(1)

    @pl.when(j == 0)
    def _():
        o_ref[...] = base_ref[...]

    rows = i * TM + lax.broadcasted_iota(jnp.int32, (TM, 1), 0)
    oh = (rows == idx_ref[...].reshape(1, TM)).astype(jnp.float32)
    o_ref[...] += jnp.dot(oh, msg_ref[...],
                          preferred_element_type=jnp.float32)


def _tc_scatter(scat_idx2d, msg, base):
    return pl.pallas_call(
        _tc_scatter_body,
        grid=(GRID_BASE, GRID),
        in_specs=[
            pl.BlockSpec((TM, 1), lambda i, j: (j, 0)),
            pl.BlockSpec((TM, D), lambda i, j: (j, 0)),
            pl.BlockSpec((TM, D), lambda i, j: (i, 0)),
        ],
        out_specs=pl.BlockSpec((TM, D), lambda i, j: (i, 0)),
        out_shape=jax.ShapeDtypeStruct((EP, D), jnp.float32),
    )(scat_idx2d, msg, base)


def kernel(x, dependency_triples, W_self, b_self, W_dep, b_dep):
    dep = dependency_triples[:, 0].astype(jnp.int32)
    lab = dependency_triples[:, 1].astype(jnp.int32)
    gov = dependency_triples[:, 2].astype(jnp.int32)
    pad = EP - E
    zpad = jnp.zeros((pad,), jnp.int32)
    trash = N + jnp.arange(pad, dtype=jnp.int32)
    gather_idx = jnp.concatenate([gov, zpad, dep, zpad]).reshape(NW, KG, CH)
    lab2 = jnp.concatenate([lab, zpad, lab, zpad]).reshape(-1, 1)
    scat_idx = jnp.concatenate([dep, trash, gov, trash])
    x_pad = jnp.concatenate([x, jnp.zeros((EP - N, D), x.dtype)])
    wt2d = jnp.transpose(W_dep.reshape(2, L, D, D), (0, 3, 1, 2)
                         ).reshape(2, D, L * D)
    wt2d = jnp.concatenate([wt2d, jnp.zeros_like(wt2d)], axis=1)
    x128 = jnp.concatenate([x, jnp.zeros_like(x)], axis=1)
    b2 = b_dep.reshape(2, L, D)
    wself_t = W_self.T
    bself2d = b_self.reshape(1, D)

    sc_gather, sc_scatter = _sc_kernels()
    g = sc_gather(x128, gather_idx)
    msg, base = _tc_compute(lab2, g, x_pad, wt2d, b2, wself_t, bself2d)
    del sc_scatter
    out = _tc_scatter(scat_idx.reshape(-1, 1), msg, base)
    return out[:N]
